# baseline (speedup 1.0000x reference)
import functools
import math

import numpy as np
import jax
import jax.numpy as jnp
from jax import lax
from jax.experimental import pallas as pl
from jax.experimental.pallas import tpu as pltpu
from jax.experimental.pallas import tpu_sc as plsc

_D = 1024
_MXLEN = 8192
_MAX_TIMESCALE = 10000.0
_SCALE = math.sqrt(_D)

_NC = 2
_NS = 16
_NW = _NC * _NS

_CHUNK = 32
_NCHUNK = 4
_HALF_PER_W = _NCHUNK * _CHUNK
_ROWS_PER_W = 2 * _HALF_PER_W


def _pe_table_np(seq):
    inc = math.log(_MAX_TIMESCALE) / _D
    inv_timescales = np.exp(
        np.arange(0, _D, 2, dtype=np.float32) * -inc).astype(np.float32)
    position = np.arange(0, seq, dtype=np.float32)[:, None]
    pe = np.zeros((seq, _D), dtype=np.float32)
    pe[:, 0::2] = np.sin(position * inv_timescales)
    pe[:, 1::2] = np.cos(position * inv_timescales)
    return pe


_PE = _pe_table_np(_MXLEN)


def _sc_gather_half(table, idxh):
    n_rows = _NW * _HALF_PER_W
    mesh = plsc.VectorSubcoreMesh(core_axis_name="c", subcore_axis_name="s")

    @functools.partial(
        pl.kernel,
        mesh=mesh,
        out_type=jax.ShapeDtypeStruct((n_rows, _D), jnp.float32),
        scratch_types=[
            pltpu.VMEM((_NCHUNK, _CHUNK), jnp.int32),
            pltpu.VMEM((_CHUNK, _D), jnp.float32),
            pltpu.VMEM((_CHUNK, _D), jnp.float32),
            pltpu.SemaphoreType.DMA,
            pltpu.SemaphoreType.DMA,
            pltpu.SemaphoreType.DMA,
            pltpu.SemaphoreType.DMA,
        ],
    )
    def k(table_hbm, idx_hbm, out_hbm, idx_v, rows0, rows1, g0, g1, w0, w1):
        wid = lax.axis_index("s") * _NC + lax.axis_index("c")
        base = wid * _HALF_PER_W
        rows = (rows0, rows1)
        gsem = (g0, g1)
        wsem = (w0, w1)
        pltpu.sync_copy(idx_hbm.at[wid], idx_v)
        gcp = [None, None]
        wcp = [None, None]
        gcp[0] = pltpu.async_copy(table_hbm.at[idx_v.at[0]], rows[0], gsem[0])
        for c in range(_NCHUNK):
            b = c % 2
            nb = 1 - b
            if c + 1 < _NCHUNK:
                if wcp[nb] is not None:
                    wcp[nb].wait()
                gcp[nb] = pltpu.async_copy(
                    table_hbm.at[idx_v.at[c + 1]], rows[nb], gsem[nb])
            gcp[b].wait()
            wcp[b] = pltpu.async_copy(
                rows[b], out_hbm.at[pl.ds(base + c * _CHUNK, _CHUNK)], wsem[b])
        wcp[0].wait()
        wcp[1].wait()

    return k(table, idxh)


def _fma_first(g_ref, pe_ref, o_ref):
    o_ref[...] = g_ref[...] * _SCALE + pe_ref[...]


def _fma_second(g_ref, pe_ref, prev_ref, o_ref):
    del prev_ref
    o_ref[...] = g_ref[...] * _SCALE + pe_ref[...]


def kernel(x, table):
    batch, seq = x.shape
    n_rows = batch * seq
    assert n_rows == _NW * _ROWS_PER_W
    pos_blk_per_seq = seq // _ROWS_PER_W

    idx3 = x.reshape(_NW, 2, _NCHUNK, _CHUNK)
    g0 = _sc_gather_half(table, idx3[:, 0])
    g1 = _sc_gather_half(table, idx3[:, 1])

    pe = jnp.asarray(_PE[:seq])
    nblk = _NW

    def specs(h):
        return dict(
            grid=(nblk,),
            in_specs=[
                pl.BlockSpec((_HALF_PER_W, _D), lambda i: (i, 0)),
                pl.BlockSpec(
                    (_HALF_PER_W, _D),
                    lambda i: ((i % pos_blk_per_seq) * 2 + h, 0)),
            ],
            out_specs=pl.BlockSpec((_HALF_PER_W, _D),
                                   lambda i: (2 * i + h, 0)),
            out_shape=jax.ShapeDtypeStruct((n_rows, _D), jnp.float32),
        )

    s0 = specs(0)
    out0 = pl.pallas_call(_fma_first, **s0)(g0, pe)

    s1 = specs(1)
    s1["in_specs"].append(pl.BlockSpec(memory_space=pl.ANY))
    out = pl.pallas_call(
        _fma_second, input_output_aliases={2: 0}, **s1)(g1, pe, out0)

    return out.reshape(batch, seq, _D)


# speedup vs baseline: 1.3749x; 1.3749x over previous
import functools
import math

import numpy as np
import jax
import jax.numpy as jnp
from jax import lax
from jax.experimental import pallas as pl
from jax.experimental.pallas import tpu as pltpu
from jax.experimental.pallas import tpu_sc as plsc

_D = 1024
_MXLEN = 8192
_MAX_TIMESCALE = 10000.0
_SCALE = math.sqrt(_D)

_NC = 2
_NS = 16
_NW = _NC * _NS

_CHUNK = 32
_NCHUNK = 8


def _pe_table_np(seq):
    inc = math.log(_MAX_TIMESCALE) / _D
    inv_timescales = np.exp(
        np.arange(0, _D, 2, dtype=np.float32) * -inc).astype(np.float32)
    position = np.arange(0, seq, dtype=np.float32)[:, None]
    pe = np.zeros((seq, _D), dtype=np.float32)
    pe[:, 0::2] = np.sin(position * inv_timescales)
    pe[:, 1::2] = np.cos(position * inv_timescales)
    return pe


_PE = _pe_table_np(_MXLEN)


def _sc_gather(table, idx3):
    n_rows = _NW * _NCHUNK * _CHUNK
    mesh = plsc.VectorSubcoreMesh(core_axis_name="c", subcore_axis_name="s")

    @functools.partial(
        pl.kernel,
        mesh=mesh,
        out_type=jax.ShapeDtypeStruct((n_rows, _D), jnp.float32),
        scratch_types=[
            pltpu.VMEM((_NCHUNK, _CHUNK), jnp.int32),
            pltpu.VMEM((_CHUNK, _D), jnp.float32),
            pltpu.VMEM((_CHUNK, _D), jnp.float32),
            pltpu.SemaphoreType.DMA,
            pltpu.SemaphoreType.DMA,
            pltpu.SemaphoreType.DMA,
            pltpu.SemaphoreType.DMA,
        ],
    )
    def k(table_hbm, idx_hbm, out_hbm, idx_v, rows0, rows1, g0, g1, w0, w1):
        wid = lax.axis_index("s") * _NC + lax.axis_index("c")
        base = wid * (_NCHUNK * _CHUNK)
        rows = (rows0, rows1)
        gsem = (g0, g1)
        wsem = (w0, w1)
        pltpu.sync_copy(idx_hbm.at[wid], idx_v)
        gcp = [None, None]
        wcp = [None, None]
        gcp[0] = pltpu.async_copy(table_hbm.at[idx_v.at[0]], rows[0], gsem[0])
        for c in range(_NCHUNK):
            b = c % 2
            nb = 1 - b
            if c + 1 < _NCHUNK:
                if wcp[nb] is not None:
                    wcp[nb].wait()
                gcp[nb] = pltpu.async_copy(
                    table_hbm.at[idx_v.at[c + 1]], rows[nb], gsem[nb])
            gcp[b].wait()
            wcp[b] = pltpu.async_copy(
                rows[b], out_hbm.at[pl.ds(base + c * _CHUNK, _CHUNK)], wsem[b])
        wcp[0].wait()
        wcp[1].wait()

    return k(table, idx3)


def _fma_body(g_ref, pe_ref, o_ref):
    o_ref[...] = g_ref[...] * _SCALE + pe_ref[...]


def kernel(x, table):
    batch, seq = x.shape
    n_rows = batch * seq
    assert n_rows == _NW * _NCHUNK * _CHUNK

    idx3 = x.reshape(_NW, _NCHUNK, _CHUNK)
    g = _sc_gather(table, idx3)

    pe = jnp.asarray(_PE[:seq])
    blk = 1024
    npe = seq // blk
    out = pl.pallas_call(
        _fma_body,
        grid=(npe, batch),
        in_specs=[
            pl.BlockSpec((blk, _D), lambda i, j: (j * npe + i, 0)),
            pl.BlockSpec((blk, _D), lambda i, j: (i, 0)),
        ],
        out_specs=pl.BlockSpec((blk, _D), lambda i, j: (j * npe + i, 0)),
        out_shape=jax.ShapeDtypeStruct((n_rows, _D), jnp.float32),
    )(g, pe)

    return out.reshape(batch, seq, _D)


# speedup vs baseline: 1.4079x; 1.0240x over previous
import functools
import math

import numpy as np
import jax
import jax.numpy as jnp
from jax import lax
from jax.experimental import pallas as pl
from jax.experimental.pallas import tpu as pltpu
from jax.experimental.pallas import tpu_sc as plsc

_D = 1024
_MXLEN = 8192
_MAX_TIMESCALE = 10000.0
_SCALE = math.sqrt(_D)

_NC = 2
_NS = 16
_NW = _NC * _NS

_CHUNK = 32
_NCHUNK = 8


def _pe_table_np(seq):
    inc = math.log(_MAX_TIMESCALE) / _D
    inv_timescales = np.exp(
        np.arange(0, _D, 2, dtype=np.float32) * -inc).astype(np.float32)
    position = np.arange(0, seq, dtype=np.float32)[:, None]
    pe = np.zeros((seq, _D), dtype=np.float32)
    pe[:, 0::2] = np.sin(position * inv_timescales)
    pe[:, 1::2] = np.cos(position * inv_timescales)
    return pe


_PE = _pe_table_np(_MXLEN)


def _sc_gather(table, idx3):
    n_rows = _NW * _NCHUNK * _CHUNK
    mesh = plsc.VectorSubcoreMesh(core_axis_name="c", subcore_axis_name="s")

    @functools.partial(
        pl.kernel,
        mesh=mesh,
        out_type=jax.ShapeDtypeStruct((n_rows, _D), jnp.float32),
        scratch_types=[
            pltpu.VMEM((_NCHUNK, _CHUNK), jnp.int32),
            pltpu.VMEM((_CHUNK, _D), jnp.float32),
            pltpu.VMEM((_CHUNK, _D), jnp.float32),
            pltpu.SemaphoreType.DMA,
            pltpu.SemaphoreType.DMA,
            pltpu.SemaphoreType.DMA,
            pltpu.SemaphoreType.DMA,
        ],
    )
    def k(table_hbm, idx_hbm, out_hbm, idx_v, rows0, rows1, g0, g1, w0, w1):
        wid = lax.axis_index("s") * _NC + lax.axis_index("c")
        base = wid * (_NCHUNK * _CHUNK)
        rows = (rows0, rows1)
        gsem = (g0, g1)
        wsem = (w0, w1)
        pltpu.sync_copy(idx_hbm.at[wid], idx_v)
        gcp = [None, None]
        wcp = [None, None]
        gcp[0] = pltpu.async_copy(table_hbm.at[idx_v.at[0]], rows[0], gsem[0])
        for c in range(_NCHUNK):
            b = c % 2
            nb = 1 - b
            if c + 1 < _NCHUNK:
                if wcp[nb] is not None:
                    wcp[nb].wait()
                gcp[nb] = pltpu.async_copy(
                    table_hbm.at[idx_v.at[c + 1]], rows[nb], gsem[nb])
            gcp[b].wait()
            wcp[b] = pltpu.async_copy(
                rows[b], out_hbm.at[pl.ds(base + c * _CHUNK, _CHUNK)], wsem[b])
        wcp[0].wait()
        wcp[1].wait()

    return k(table, idx3)


def _fma_body(g_ref, pe_ref, o_ref):
    o_ref[...] = g_ref[...] * _SCALE + pe_ref[...]


def kernel(x, table):
    batch, seq = x.shape
    n_rows = batch * seq
    assert n_rows == _NW * _NCHUNK * _CHUNK

    idx3 = x.reshape(_NW, _NCHUNK, _CHUNK)
    g = _sc_gather(table, idx3)

    pe = jnp.asarray(_PE[:seq])
    blk = 2048
    npe = seq // blk
    out = pl.pallas_call(
        _fma_body,
        grid=(npe, batch),
        in_specs=[
            pl.BlockSpec((blk, _D), lambda i, j: (j * npe + i, 0)),
            pl.BlockSpec((blk, _D), lambda i, j: (i, 0)),
        ],
        out_specs=pl.BlockSpec((blk, _D), lambda i, j: (j * npe + i, 0)),
        out_shape=jax.ShapeDtypeStruct((n_rows, _D), jnp.float32),
    )(g, pe)

    return out.reshape(batch, seq, _D)


# speedup vs baseline: 1.4280x; 1.0142x over previous
import functools
import math

import numpy as np
import jax
import jax.numpy as jnp
from jax import lax
from jax.experimental import pallas as pl
from jax.experimental.pallas import tpu as pltpu
from jax.experimental.pallas import tpu_sc as plsc

_D = 1024
_MXLEN = 8192
_MAX_TIMESCALE = 10000.0
_SCALE = math.sqrt(_D)

_NC = 2
_NS = 16
_NW = _NC * _NS

_CHUNK = 32
_NCHUNK = 8


def _pe_table_np(seq):
    inc = math.log(_MAX_TIMESCALE) / _D
    inv_timescales = np.exp(
        np.arange(0, _D, 2, dtype=np.float32) * -inc).astype(np.float32)
    position = np.arange(0, seq, dtype=np.float32)[:, None]
    pe = np.zeros((seq, _D), dtype=np.float32)
    pe[:, 0::2] = np.sin(position * inv_timescales)
    pe[:, 1::2] = np.cos(position * inv_timescales)
    return pe


_PE = _pe_table_np(_MXLEN)


def _sc_gather(table, idx3):
    n_rows = _NW * _NCHUNK * _CHUNK
    mesh = plsc.VectorSubcoreMesh(core_axis_name="c", subcore_axis_name="s")

    @functools.partial(
        pl.kernel,
        mesh=mesh,
        out_type=jax.ShapeDtypeStruct((n_rows, _D), jnp.float32),
        scratch_types=[
            pltpu.VMEM((_NCHUNK, _CHUNK), jnp.int32),
            pltpu.VMEM((_CHUNK, _D), jnp.float32),
            pltpu.VMEM((_CHUNK, _D), jnp.float32),
            pltpu.VMEM((_CHUNK, _D), jnp.float32),
            pltpu.SemaphoreType.DMA,
            pltpu.SemaphoreType.DMA,
            pltpu.SemaphoreType.DMA,
            pltpu.SemaphoreType.DMA,
            pltpu.SemaphoreType.DMA,
            pltpu.SemaphoreType.DMA,
        ],
    )
    def k(table_hbm, idx_hbm, out_hbm, idx_v,
          rows0, rows1, rows2, g0, g1, g2, w0, w1, w2):
        wid = lax.axis_index("s") * _NC + lax.axis_index("c")
        base = wid * (_NCHUNK * _CHUNK)
        rows = (rows0, rows1, rows2)
        gsem = (g0, g1, g2)
        wsem = (w0, w1, w2)
        pltpu.sync_copy(idx_hbm.at[wid], idx_v)
        gcp = [None, None, None]
        wcp = [None, None, None]
        gcp[0] = pltpu.async_copy(table_hbm.at[idx_v.at[0]], rows[0], gsem[0])
        gcp[1] = pltpu.async_copy(table_hbm.at[idx_v.at[1]], rows[1], gsem[1])
        for c in range(_NCHUNK):
            b = c % 3
            if c + 2 < _NCHUNK:
                tb = (c + 2) % 3
                if wcp[tb] is not None:
                    wcp[tb].wait()
                gcp[tb] = pltpu.async_copy(
                    table_hbm.at[idx_v.at[c + 2]], rows[tb], gsem[tb])
            gcp[b].wait()
            wcp[b] = pltpu.async_copy(
                rows[b], out_hbm.at[pl.ds(base + c * _CHUNK, _CHUNK)], wsem[b])
        for b in range(3):
            if wcp[b] is not None:
                wcp[b].wait()

    return k(table, idx3)


def _fma_body(g_ref, pe_ref, o_ref):
    o_ref[...] = g_ref[...] * _SCALE + pe_ref[...]


def kernel(x, table):
    batch, seq = x.shape
    n_rows = batch * seq
    assert n_rows == _NW * _NCHUNK * _CHUNK

    idx3 = x.reshape(_NW, _NCHUNK, _CHUNK)
    g = _sc_gather(table, idx3)

    pe = jnp.asarray(_PE[:seq])
    blk = 2048
    npe = seq // blk
    out = pl.pallas_call(
        _fma_body,
        grid=(npe, batch),
        in_specs=[
            pl.BlockSpec((blk, _D), lambda i, j: (j * npe + i, 0)),
            pl.BlockSpec((blk, _D), lambda i, j: (i, 0)),
        ],
        out_specs=pl.BlockSpec((blk, _D), lambda i, j: (j * npe + i, 0)),
        out_shape=jax.ShapeDtypeStruct((n_rows, _D), jnp.float32),
    )(g, pe)

    return out.reshape(batch, seq, _D)
